# baseline (device time: 22133 ns/iter reference)
import jax
import jax.numpy as jnp
from jax import lax
from jax.experimental import pallas as pl
from jax.experimental.pallas import tpu as pltpu

N_DEV = 8
BLK = 256


def kernel(x, w_mat):
    m_per, k = x.shape
    _, n = w_mat.shape
    assert n == N_DEV * BLK and m_per == BLK

    def body(x_ref, w_ref, out_ref, send_buf, recv_buf, send_sems, recv_sems):
        my = lax.axis_index("i")

        bar = pltpu.get_barrier_semaphore()
        for d in range(1, N_DEV):
            pl.semaphore_signal(
                bar, inc=1,
                device_id=((my + d) % N_DEV,),
                device_id_type=pl.DeviceIdType.MESH,
            )
        pl.semaphore_wait(bar, N_DEV - 1)

        x_val = x_ref[...]
        rdmas = []
        for d in range(N_DEV):
            p = (my + d) % N_DEV
            w_col = w_ref[:, pl.ds(p * BLK, BLK)]
            y = jnp.dot(x_val, w_col, preferred_element_type=jnp.float32)
            z = jnp.maximum(y, 0.0).astype(jnp.bfloat16)
            if d == 0:
                out_ref[pl.ds(my * BLK, BLK), :] = z
            else:
                send_buf[d] = z
                rdma = pltpu.make_async_remote_copy(
                    src_ref=send_buf.at[d],
                    dst_ref=recv_buf.at[d],
                    send_sem=send_sems.at[d],
                    recv_sem=recv_sems.at[d],
                    device_id=(p,),
                    device_id_type=pl.DeviceIdType.MESH,
                )
                rdma.start()
                rdmas.append(rdma)

        for d in range(1, N_DEV):
            src = (my - d) % N_DEV
            rdmas[d - 1].wait_recv()
            out_ref[pl.ds(src * BLK, BLK), :] = recv_buf[d]
        for r in rdmas:
            r.wait_send()

    return pl.pallas_call(
        body,
        out_shape=jax.ShapeDtypeStruct((n, BLK), jnp.bfloat16),
        in_specs=[
            pl.BlockSpec(memory_space=pltpu.VMEM),
            pl.BlockSpec(memory_space=pltpu.VMEM),
        ],
        out_specs=pl.BlockSpec(memory_space=pltpu.VMEM),
        scratch_shapes=[
            pltpu.VMEM((N_DEV, BLK, BLK), jnp.bfloat16),
            pltpu.VMEM((N_DEV, BLK, BLK), jnp.bfloat16),
            pltpu.SemaphoreType.DMA((N_DEV,)),
            pltpu.SemaphoreType.DMA((N_DEV,)),
        ],
        compiler_params=pltpu.CompilerParams(collective_id=0),
    )(x, w_mat)
